# baseline (speedup 1.0000x reference)
import functools

import jax
import jax.numpy as jnp
from jax.experimental import pallas as pl

DIM = 4096
NUM_EXPERTS = 64
BLOCK_T = 1024


def _router_body(x_ref, w_ref, topi_ref, wts_ref, pri_ref, *, block_t):
    logits = jax.lax.dot_general(
        x_ref[...], w_ref[...],
        dimension_numbers=(((1,), (1,)), ((), ())),
        preferred_element_type=jnp.float32,
    )
    m = jnp.max(logits, axis=1, keepdims=True)
    idx = jnp.argmax(logits, axis=1)
    s = jnp.sum(jnp.exp(logits - m), axis=1, keepdims=True)
    topi_ref[...] = idx[:, None].astype(jnp.int32)
    wts_ref[...] = 1.0 / s
    row0 = pl.program_id(0) * block_t
    rows = row0 + jax.lax.broadcasted_iota(jnp.int32, (block_t, 1), 0)
    pri_ref[...] = -rows.astype(jnp.float32)


@jax.jit
def kernel(x, gate_w):
    t = x.shape[0]
    grid = (t // BLOCK_T,)
    topi, wts, pri = pl.pallas_call(
        functools.partial(_router_body, block_t=BLOCK_T),
        grid=grid,
        in_specs=[
            pl.BlockSpec((BLOCK_T, DIM), lambda i: (i, 0)),
            pl.BlockSpec((NUM_EXPERTS, DIM), lambda i: (0, 0)),
        ],
        out_specs=[
            pl.BlockSpec((BLOCK_T, 1), lambda i: (i, 0)),
            pl.BlockSpec((BLOCK_T, 1), lambda i: (i, 0)),
            pl.BlockSpec((BLOCK_T, 1), lambda i: (i, 0)),
        ],
        out_shape=[
            jax.ShapeDtypeStruct((t, 1), jnp.int32),
            jax.ShapeDtypeStruct((t, 1), jnp.float32),
            jax.ShapeDtypeStruct((t, 1), jnp.float32),
        ],
    )(x, gate_w)
    return (topi, wts, pri.reshape(t))
